# baseline (speedup 1.0000x reference)
import functools
import math

import jax
import jax.numpy as jnp
from jax import lax
from jax.experimental import pallas as pl
from jax.experimental.pallas import tpu as pltpu
from jax.experimental.pallas import tpu_sc as plsc

_LANES = 16


def _pos_encoding(T_len, d_model, dtype):
    positions = jnp.arange(T_len, dtype=dtype)[:, None]
    i = jnp.arange(0, d_model, 2, dtype=dtype)
    denominator = jnp.exp(i / d_model * math.log(10000.0))
    pe = jnp.zeros((T_len, d_model), dtype=dtype)
    pe = pe.at[:, 0::2].set(jnp.sin(positions / denominator))
    pe = pe.at[:, 1::2].set(jnp.cos(positions / denominator))
    return pe


def _make_sc_kernel(N, V, D, T, scale):
    try:
        info = plsc.get_sparse_core_info()
        NC, NS = info.num_cores, info.num_subcores
    except ValueError:
        NC, NS = 2, 16
    NW = NC * NS
    assert N % (NW * T) == 0, (N, NW, T)
    seqs_per_w = N // (NW * T)
    rows_per_w = seqs_per_w * T
    c0 = 128 if T > 128 else T
    c1 = T - c0
    mesh = plsc.VectorSubcoreMesh(
        core_axis_name="c",
        subcore_axis_name="s",
        num_cores=NC,
        num_subcores=NS,
    )

    scratch = [
        pltpu.VMEM((c0,), jnp.int32),
        pltpu.VMEM((c1,), jnp.int32) if c1 else None,
        pltpu.VMEM((T, D), jnp.float32),
        pltpu.VMEM((T, D), jnp.float32),
        pltpu.SemaphoreType.DMA,
    ]
    scratch = [s for s in scratch if s is not None]

    @functools.partial(
        pl.kernel,
        out_type=jax.ShapeDtypeStruct((N, D), jnp.float32),
        mesh=mesh,
        scratch_types=scratch,
        compiler_params=pltpu.CompilerParams(use_tc_tiling_on_sc=False),
    )
    def k(x_hbm, we_hbm, pe_hbm, out_hbm, idx_a, idx_b, rows_v, pe_v, sem):
        wid = lax.axis_index("s") * NC + lax.axis_index("c")
        base = wid * rows_per_w
        pltpu.sync_copy(pe_hbm, pe_v)

        def seq_body(g, carry):
            row0 = base + g * T
            pltpu.sync_copy(x_hbm.at[pl.ds(row0, c0)], idx_a)
            pltpu.sync_copy(x_hbm.at[pl.ds(row0 + c0, c1)], idx_b)
            cp0 = pltpu.async_copy(we_hbm.at[idx_a], rows_v.at[pl.ds(0, c0)], sem)
            cp1 = pltpu.async_copy(we_hbm.at[idx_b], rows_v.at[pl.ds(c0, c1)], sem)
            cp0.wait()
            cp1.wait()

            def row_body(t, c):
                for j in range(D // _LANES):
                    sl = pl.ds(j * _LANES, _LANES)
                    rows_v[t, sl] = rows_v[t, sl] * scale + pe_v[t, sl]
                return c

            lax.fori_loop(0, T, row_body, 0, unroll=2)
            pltpu.sync_copy(rows_v, out_hbm.at[pl.ds(row0, T)])
            return carry

        lax.fori_loop(0, seqs_per_w, seq_body, 0)

    return k


@jax.jit
def kernel(x, We):
    B, T = x.shape
    V, D = We.shape
    scale = math.sqrt(D)
    pe = _pos_encoding(T, D, jnp.float32)
    x_flat = x.reshape(B * T)
    sc = _make_sc_kernel(B * T, V, D, T, scale)
    out = sc(x_flat, We, pe)
    return out.reshape(B, T, D)


# speedup vs baseline: 1.1852x; 1.1852x over previous
import functools
import math

import jax
import jax.numpy as jnp
from jax import lax
from jax.experimental import pallas as pl
from jax.experimental.pallas import tpu as pltpu
from jax.experimental.pallas import tpu_sc as plsc

_LANES = 16
_C = 128
_NBUF = 4


def _pos_encoding(T_len, d_model, dtype):
    positions = jnp.arange(T_len, dtype=dtype)[:, None]
    i = jnp.arange(0, d_model, 2, dtype=dtype)
    denominator = jnp.exp(i / d_model * math.log(10000.0))
    pe = jnp.zeros((T_len, d_model), dtype=dtype)
    pe = pe.at[:, 0::2].set(jnp.sin(positions / denominator))
    pe = pe.at[:, 1::2].set(jnp.cos(positions / denominator))
    return pe


def _make_sc_kernel(N, V, D, T, scale):
    try:
        info = plsc.get_sparse_core_info()
        NC, NS = info.num_cores, info.num_subcores
    except ValueError:
        NC, NS = 2, 16
    NW = NC * NS
    C = _C
    assert N % (NW * T) == 0 and (N // NW) % C == 0, (N, NW, T)
    rows_per_w = N // NW
    ncheck = rows_per_w // C
    assert ncheck % _NBUF == 0
    mesh = plsc.VectorSubcoreMesh(
        core_axis_name="c",
        subcore_axis_name="s",
        num_cores=NC,
        num_subcores=NS,
    )

    scratch = [
        pltpu.VMEM((ncheck, C), jnp.int32),
        pltpu.VMEM((2 * T, D), jnp.float32),
        pltpu.VMEM((_NBUF, C, D), jnp.float32),
        [pltpu.SemaphoreType.DMA] * _NBUF,
        [pltpu.SemaphoreType.DMA] * _NBUF,
    ]

    @functools.partial(
        pl.kernel,
        out_type=jax.ShapeDtypeStruct((N, D), jnp.float32),
        mesh=mesh,
        scratch_types=scratch,
        compiler_params=pltpu.CompilerParams(use_tc_tiling_on_sc=False),
    )
    def k(x_hbm, we_hbm, pe2_hbm, out_hbm, idx_v, pe_v, rows_v, sem_g, sem_o):
        wid = lax.axis_index("s") * NC + lax.axis_index("c")
        base = wid * rows_per_w
        pltpu.sync_copy(pe2_hbm, pe_v)
        pltpu.sync_copy(x_hbm.at[pl.ds(wid * ncheck, ncheck)], idx_v)

        def gather(g, b):
            return pltpu.make_async_copy(
                we_hbm.at[idx_v.at[g]], rows_v.at[b], sem_g[b]
            )

        def writeout(g, b):
            return pltpu.make_async_copy(
                rows_v.at[b], out_hbm.at[pl.ds(base + g * C, C)], sem_o[b]
            )

        gather(0, 0).start()
        gather(1, 1).start()

        def outer(gg, carry):
            for b in range(_NBUF):
                g = gg * _NBUF + b
                bp = (b + 2) % _NBUF

                @pl.when(g + 2 < ncheck)
                def _():
                    @pl.when(g >= 2)
                    def _():
                        writeout(g - 2, bp).wait()

                    gather(g + 2, bp).start()

                gather(g, b).wait()
                t0 = (g * C) % T

                def row_body(r, c):
                    for j in range(D // _LANES):
                        sl = pl.ds(j * _LANES, _LANES)
                        rows_v[b, r, sl] = rows_v[b, r, sl] * scale + pe_v[t0 + r, sl]
                    return c

                lax.fori_loop(0, C, row_body, 0, unroll=4)
                writeout(g, b).start()
            return carry

        lax.fori_loop(0, ncheck // _NBUF, outer, 0)
        for b in range(_NBUF):
            writeout(ncheck - _NBUF + b, b).wait()

    return k


@jax.jit
def kernel(x, We):
    B, T = x.shape
    V, D = We.shape
    scale = math.sqrt(D)
    pe = _pos_encoding(T, D, jnp.float32)
    pe2 = jnp.concatenate([pe, pe], axis=0)
    x2d = x.reshape((B * T) // _C, _C)
    sc = _make_sc_kernel(B * T, V, D, T, scale)
    out = sc(x2d, We, pe2)
    return out.reshape(B, T, D)
